# baseline (speedup 1.0000x reference)
import functools

import jax
import jax.numpy as jnp
from jax import lax
from jax.experimental import pallas as pl
from jax.experimental.pallas import tpu as pltpu
from jax.experimental.pallas import tpu_sc as plsc

_NBUF = 4
_SW = 1024


def _sc_shuffle(cache4, tok_flat, idx_arr):
    H, B, D, S = cache4.shape
    NW = 32
    PPW = (H * B) // NW
    DT = D // 8
    NSH = S // _SW
    CPP = DT * NSH
    ngrp = PPW * CPP // _NBUF

    mesh = plsc.VectorSubcoreMesh(core_axis_name="c", subcore_axis_name="s")

    @functools.partial(
        pl.kernel,
        mesh=mesh,
        out_type=jax.ShapeDtypeStruct((B, H, D, S), jnp.float32),
        scratch_types=[
            pltpu.VMEM((_NBUF, 8, _SW), jnp.float32),
            pltpu.VMEM_SHARED((2, B * H * D), jnp.float32),
            pltpu.VMEM((16,), jnp.int32),
        ] + [pltpu.SemaphoreType.DMA] * (2 * _NBUF + 1),
    )
    def sc_update(idx_hbm, tok_hbm, c_hbm, o_hbm, bufs, kbuf, idx_v, *sems):
        semr = sems[:_NBUF]
        semw = sems[_NBUF:2 * _NBUF]
        semp = sems[2 * _NBUF]
        sid = lax.axis_index("s")
        w = sid * 2 + lax.axis_index("c")

        pltpu.sync_copy(idx_hbm, idx_v)
        t0 = idx_v[...][0]

        @pl.when(sid == 0)
        def _():
            pltpu.sync_copy(tok_hbm, kbuf.at[0])

        plsc.subcore_barrier()

        def coords(t):
            p = w * PPW + t // CPP
            c = t % CPP
            return p // B, p % B, (c // NSH) * 8, (c % NSH) * _SW

        def group(g, carry):
            for j in range(_NBUF):
                h, b, d0, s0 = coords(g * _NBUF + j)

                @pl.when(g > 0)
                def _():
                    pltpu.make_async_copy(
                        bufs.at[j], o_hbm.at[0, 0, pl.ds(0, 8), pl.ds(0, _SW)],
                        semw[j]).wait()

                pltpu.async_copy(
                    c_hbm.at[h, b, pl.ds(d0, 8), pl.ds(s0, _SW)],
                    bufs.at[j], semr[j])
            for j in range(_NBUF):
                h, b, d0, s0 = coords(g * _NBUF + j)
                pltpu.make_async_copy(
                    c_hbm.at[0, 0, pl.ds(0, 8), pl.ds(0, _SW)],
                    bufs.at[j], semr[j]).wait()
                rel = t0 - s0

                @pl.when((rel >= 0) & (rel < _SW))
                def _():
                    pltpu.async_copy(
                        kbuf.at[0, pl.ds((b * H + h) * D + d0, 8)],
                        bufs.at[j, :, rel], semp).wait()

                pltpu.async_copy(
                    bufs.at[j], o_hbm.at[b, h, pl.ds(d0, 8), pl.ds(s0, _SW)],
                    semw[j])
            return carry

        lax.fori_loop(0, ngrp, group, 0)
        for j in range(_NBUF):
            pltpu.make_async_copy(
                bufs.at[j], o_hbm.at[0, 0, pl.ds(0, 8), pl.ds(0, _SW)],
                semw[j]).wait()

    return sc_update(idx_arr, tok_flat, cache4)


def _tc_body(idx_ref, c_ref, tok_ref, o_ref):
    s = idx_ref[0]
    base = pl.multiple_of((s // 128) * 128, 128)
    o_ref[...] = c_ref[...]
    sub = c_ref[:, pl.ds(base, 128)]
    col = jax.lax.broadcasted_iota(jnp.int32, sub.shape, 1) + base
    o_ref[:, pl.ds(base, 128)] = jnp.where(col == s, tok_ref[...], sub)


def _tc_shuffle(cache4, tok41, idx):
    H, B, D, S = cache4.shape
    cache_spec = pl.BlockSpec((None, None, D, S), lambda h, b, i: (h, b, 0, 0))
    tok_spec = pl.BlockSpec((None, None, D, 1), lambda h, b, i: (b, h, 0, 0))
    out_spec = pl.BlockSpec((None, None, D, S), lambda h, b, i: (b, h, 0, 0))

    return pl.pallas_call(
        _tc_body,
        grid_spec=pltpu.PrefetchScalarGridSpec(
            num_scalar_prefetch=1,
            grid=(H, B),
            in_specs=[cache_spec, tok_spec],
            out_specs=out_spec,
        ),
        out_shape=jax.ShapeDtypeStruct((B, H, D, S), jnp.float32),
        compiler_params=pltpu.CompilerParams(
            dimension_semantics=("arbitrary", "arbitrary"),
        ),
    )(idx, cache4, tok41)


def kernel(key, value, cached_ar_key, cached_ar_value, cache_ar_index):
    S, H, B, D = cached_ar_key.shape

    ck4 = jnp.transpose(cached_ar_key, (1, 2, 3, 0))
    cv4 = jnp.transpose(cached_ar_value, (1, 2, 3, 0))
    idx = jnp.clip(jnp.asarray(cache_ar_index, jnp.int32), 0, S - 1)
    idx_arr = jnp.full((16,), idx)

    out_v = _sc_shuffle(cv4, value.reshape(B * H * D), idx_arr)
    out_k = _tc_shuffle(ck4, key.reshape(B, H, D, 1), idx.reshape(1))

    return (jnp.transpose(out_k, (0, 3, 1, 2)),
            jnp.transpose(out_v, (0, 3, 1, 2)))


# speedup vs baseline: 1.0032x; 1.0032x over previous
import functools

import jax
import jax.numpy as jnp
from jax import lax
from jax.experimental import pallas as pl
from jax.experimental.pallas import tpu as pltpu
from jax.experimental.pallas import tpu_sc as plsc

_NBUF = 4
_SW = 1024


def _sc_shuffle(cache4, tok_flat, idx_arr):
    H, B, D, S = cache4.shape
    NW = 32
    PPW = (H * B) // NW
    DT = D // 8
    NSH = S // _SW
    CPP = DT * NSH
    ngrp = PPW * CPP // _NBUF

    mesh = plsc.VectorSubcoreMesh(core_axis_name="c", subcore_axis_name="s")

    @functools.partial(
        pl.kernel,
        mesh=mesh,
        out_type=jax.ShapeDtypeStruct((B, H, D, S), jnp.float32),
        scratch_types=[
            pltpu.VMEM((_NBUF, 8, _SW), jnp.float32),
            pltpu.VMEM_SHARED((2, B * H * D), jnp.float32),
            pltpu.VMEM((16,), jnp.int32),
        ] + [pltpu.SemaphoreType.DMA] * (2 * _NBUF + 1),
    )
    def sc_update(idx_hbm, tok_hbm, c_hbm, o_hbm, bufs, kbuf, idx_v, *sems):
        semr = sems[:_NBUF]
        semw = sems[_NBUF:2 * _NBUF]
        semp = sems[2 * _NBUF]
        sid = lax.axis_index("s")
        w = sid * 2 + lax.axis_index("c")

        pltpu.sync_copy(idx_hbm, idx_v)
        t0 = idx_v[...][0]

        @pl.when(sid == 0)
        def _():
            pltpu.sync_copy(tok_hbm, kbuf.at[0])

        plsc.subcore_barrier()

        def coords(t):
            p = w * PPW + t // CPP
            c = t % CPP
            return p // B, p % B, (c // NSH) * 8, (c % NSH) * _SW

        def group(g, carry):
            for j in range(_NBUF):
                h, b, d0, s0 = coords(g * _NBUF + j)

                @pl.when(g > 0)
                def _():
                    pltpu.make_async_copy(
                        bufs.at[j], o_hbm.at[0, 0, pl.ds(0, 8), pl.ds(0, _SW)],
                        semw[j]).wait()

                pltpu.async_copy(
                    c_hbm.at[h, b, pl.ds(d0, 8), pl.ds(s0, _SW)],
                    bufs.at[j], semr[j])
            for j in range(_NBUF):
                h, b, d0, s0 = coords(g * _NBUF + j)
                pltpu.make_async_copy(
                    c_hbm.at[0, 0, pl.ds(0, 8), pl.ds(0, _SW)],
                    bufs.at[j], semr[j]).wait()
                rel = t0 - s0

                @pl.when((rel >= 0) & (rel < _SW))
                def _():
                    pltpu.async_copy(
                        kbuf.at[0, pl.ds((b * H + h) * D + d0, 8)],
                        bufs.at[j, :, rel], semp).wait()

                pltpu.async_copy(
                    bufs.at[j], o_hbm.at[b, h, pl.ds(d0, 8), pl.ds(s0, _SW)],
                    semw[j])
            return carry

        lax.fori_loop(0, ngrp, group, 0)
        for j in range(_NBUF):
            pltpu.make_async_copy(
                bufs.at[j], o_hbm.at[0, 0, pl.ds(0, 8), pl.ds(0, _SW)],
                semw[j]).wait()

    return sc_update(idx_arr, tok_flat, cache4)


def _tc_body(idx_ref, c_ref, tok_ref, o_ref):
    s = idx_ref[0]
    base = pl.multiple_of((s // 128) * 128, 128)
    o_ref[...] = c_ref[...]
    sub = c_ref[:, pl.ds(base, 128)]
    col = jax.lax.broadcasted_iota(jnp.int32, sub.shape, 1) + base
    o_ref[:, pl.ds(base, 128)] = jnp.where(col == s, tok_ref[...], sub)


def _tc_shuffle(cache4, tok41, idx):
    H, B, D, S = cache4.shape
    cache_spec = pl.BlockSpec((None, None, D, S), lambda b, h, i: (h, b, 0, 0))
    tok_spec = pl.BlockSpec((None, None, D, 1), lambda b, h, i: (b, h, 0, 0))
    out_spec = pl.BlockSpec((None, None, D, S), lambda b, h, i: (b, h, 0, 0))

    return pl.pallas_call(
        _tc_body,
        grid_spec=pltpu.PrefetchScalarGridSpec(
            num_scalar_prefetch=1,
            grid=(B, H),
            in_specs=[cache_spec, tok_spec],
            out_specs=out_spec,
        ),
        out_shape=jax.ShapeDtypeStruct((B, H, D, S), jnp.float32),
        compiler_params=pltpu.CompilerParams(
            dimension_semantics=("arbitrary", "arbitrary"),
        ),
    )(idx, cache4, tok41)


def kernel(key, value, cached_ar_key, cached_ar_value, cache_ar_index):
    S, H, B, D = cached_ar_key.shape

    ck4 = jnp.transpose(cached_ar_key, (1, 2, 3, 0))
    cv4 = jnp.transpose(cached_ar_value, (1, 2, 3, 0))
    idx = jnp.clip(jnp.asarray(cache_ar_index, jnp.int32), 0, S - 1)
    idx_arr = jnp.full((16,), idx)

    out_v = _sc_shuffle(cv4, value.reshape(B * H * D), idx_arr)
    out_k = _tc_shuffle(ck4, key.reshape(B, H, D, 1), idx.reshape(1))

    return (jnp.transpose(out_k, (0, 3, 1, 2)),
            jnp.transpose(out_v, (0, 3, 1, 2)))
